# baseline (speedup 1.0000x reference)
import functools

import jax
import jax.numpy as jnp
from jax import lax
from jax.experimental import pallas as pl
from jax.experimental.pallas import tpu as pltpu
from jax.experimental.pallas import tpu_sc as plsc

WINDOW = 5
LANES = 128


@functools.cache
def _make_prep_stage(B, L, V, D):
    def prep_body(ids_ref, tab_ref, b_ref, idsimg_ref, tabpad_ref, bt_ref):
        idsimg_ref[...] = jnp.concatenate(
            [jnp.transpose(ids_ref[...]),
             jnp.zeros((B, LANES - L), jnp.int32)], axis=1)
        tab = jnp.transpose(tab_ref[...])
        npack = LANES // D
        nimg = -(-V // npack)
        rr = jax.lax.broadcasted_iota(jnp.int32, (nimg, V), 0)
        cc = jax.lax.broadcasted_iota(jnp.int32, (nimg, V), 1)
        pieces = [
            jax.lax.dot_general(
                (cc == rr * npack + j).astype(jnp.float32), tab,
                (((1,), (0,)), ((), ())),
                preferred_element_type=jnp.float32)
            for j in range(npack)
        ]
        img = jnp.concatenate(pieces, axis=1)
        tabpad_ref[...] = jnp.concatenate(
            [img, jnp.zeros((LANES - nimg, LANES), jnp.float32)], axis=0)
        bt_ref[...] = jnp.transpose(
            jnp.broadcast_to(b_ref[...].reshape(1, V), (8, V)))

    return pl.pallas_call(
        prep_body,
        out_shape=[
            jax.ShapeDtypeStruct((B, LANES), jnp.int32),
            jax.ShapeDtypeStruct((LANES, LANES), jnp.float32),
            jax.ShapeDtypeStruct((V, 8), jnp.float32),
        ],
    )


@functools.cache
def _make_sc_stage(B, L, V, D):
    info = plsc.get_sparse_core_info()
    NC, NS = info.num_cores, info.num_subcores
    NW = NC * NS
    NL = info.num_lanes
    assert B % NW == 0 and D == NL and L >= NL
    seq_per_w = B // NW
    koffs = [i * NL for i in range(L // NL)]
    if L % NL:
        koffs.append(L - NL)

    mesh = plsc.VectorSubcoreMesh(core_axis_name="c", subcore_axis_name="s")

    @functools.partial(
        pl.kernel,
        mesh=mesh,
        compiler_params=pltpu.CompilerParams(use_tc_tiling_on_sc=False,
                                             needs_layout_passes=False),
        out_type=jax.ShapeDtypeStruct((L, D, B), jnp.float32),
        scratch_types=[
            pltpu.VMEM((seq_per_w, LANES), jnp.int32),
            pltpu.VMEM((LANES, LANES), jnp.float32),
            pltpu.VMEM((seq_per_w * L, D), jnp.float32),
            pltpu.VMEM((L * D, seq_per_w), jnp.float32),
            pltpu.SemaphoreType.DMA,
        ],
    )
    def sc_kernel(ids_hbm, table_hbm, out_hbm, idx_v, tab_v, rows_v,
                  gvt_v, sem):
        wid = lax.axis_index("s") * NC + lax.axis_index("c")

        pltpu.sync_copy(ids_hbm.at[pl.ds(wid * seq_per_w, seq_per_w)], idx_v)
        pltpu.sync_copy(table_hbm, tab_v)

        lane_i = jnp.arange(NL, dtype=jnp.int32)
        zeros_i = jnp.zeros((NL,), jnp.int32)
        dvecs = [jnp.full((NL,), d, jnp.int32) for d in range(D)]
        zero = jnp.zeros((D,), jnp.float32)
        PAIR = 2

        def seq_body(i, carry):
            seqs = [i * PAIR + j for j in range(PAIR)]
            svecs = [zeros_i + s for s in seqs]

            for s in seqs:
                for koff in koffs:
                    v = idx_v[s, pl.ds(koff, NL)]
                    ri = jax.lax.shift_right_logical(v, 3)
                    li0 = jax.lax.shift_left(jnp.bitwise_and(v, 7), 4)
                    tokvec = lane_i + (s * L + koff)
                    vals = [plsc.load_gather(tab_v, [ri, li0 + d])
                            for d in range(D)]
                    for d in range(D):
                        plsc.store_scatter(rows_v, [tokvec, dvecs[d]],
                                           vals[d])

            es = [{} for _ in seqs]
            cums = [{0: zero} for _ in seqs]

            def emit(j, p):
                c = cums[j]
                if p < WINDOW:
                    g = c[p + WINDOW] - c[1]
                elif p + WINDOW > L:
                    g = c[L - 1] - c[p - WINDOW]
                else:
                    g = c[p + WINDOW] - c[p - WINDOW] - es[j][p]
                rowvec = lane_i + p * D
                plsc.store_scatter(gvt_v, [rowvec, svecs[j]], g)

            lag = WINDOW + 1
            for t in range(L):
                for j, s in enumerate(seqs):
                    e = rows_v[s * L + t, :]
                    es[j][t] = e
                    cums[j][t + 1] = cums[j][t] + e
                for j in range(PAIR):
                    p = t - lag
                    if p >= 0:
                        emit(j, p)
            for p in range(L - lag, L):
                for j in range(PAIR):
                    emit(j, p)
            return carry

        lax.fori_loop(0, seq_per_w // PAIR, seq_body, 0)

        copies = []
        for l in range(L):
            copies.append(
                pltpu.async_copy(
                    gvt_v.at[pl.ds(l * D, D)],
                    out_hbm.at[l, pl.ds(0, D),
                               pl.ds(wid * seq_per_w, seq_per_w)],
                    sem,
                )
            )
        for cp in copies:
            cp.wait()

    return sc_kernel


@functools.cache
def _make_tc_stage(B, L, V, D, LB=2):
    assert L % LB == 0

    def mm_body(x_ref, w_ref, b_ref, o_ref):
        w = w_ref[...]
        bias = b_ref[...][:, 0:1]
        for j in range(LB):
            o_ref[j] = (
                lax.dot_general(
                    w, x_ref[j],
                    (((0,), (0,)), ((), ())),
                    preferred_element_type=jnp.float32,
                )
                + bias
            )

    return pl.pallas_call(
        mm_body,
        grid=(L // LB,),
        in_specs=[
            pl.BlockSpec((LB, D, B), lambda i: (i, 0, 0)),
            pl.BlockSpec((D, V), lambda i: (0, 0)),
            pl.BlockSpec((V, 8), lambda i: (0, 0)),
        ],
        out_specs=pl.BlockSpec((LB, V, B), lambda i: (i, 0, 0)),
        out_shape=jax.ShapeDtypeStruct((L, V, B), jnp.float32),
    )


def kernel(ids, emb_table, W, b):
    B, L = ids.shape
    V, D = emb_table.shape
    ids = ids.astype(jnp.int32)
    ids_img, tab_pad, b_t = _make_prep_stage(B, L, V, D)(
        jnp.transpose(ids), jnp.transpose(emb_table), b)
    grouped_t = _make_sc_stage(B, L, V, D)(ids_img, tab_pad)
    out_t = _make_tc_stage(B, L, V, D)(grouped_t, jnp.transpose(W), b_t)
    return jnp.transpose(out_t, (2, 0, 1))


# speedup vs baseline: 1.0097x; 1.0097x over previous
import functools

import jax
import jax.numpy as jnp
from jax import lax
from jax.experimental import pallas as pl
from jax.experimental.pallas import tpu as pltpu
from jax.experimental.pallas import tpu_sc as plsc

WINDOW = 5
LANES = 128


@functools.cache
def _make_prep_stage(B, L, V, D):
    def prep_body(ids_ref, tab_ref, b_ref, idsimg_ref, tabpad_ref, bt_ref):
        idsimg_ref[...] = jnp.concatenate(
            [jnp.transpose(ids_ref[...]),
             jnp.zeros((B, LANES - L), jnp.int32)], axis=1)
        tabpad_ref[...] = jnp.concatenate(
            [jnp.transpose(tab_ref[...]),
             jnp.zeros((V, LANES - D), jnp.float32)], axis=1)
        bt_ref[...] = jnp.transpose(
            jnp.broadcast_to(b_ref[...].reshape(1, V), (8, V)))

    return pl.pallas_call(
        prep_body,
        out_shape=[
            jax.ShapeDtypeStruct((B, LANES), jnp.int32),
            jax.ShapeDtypeStruct((V, LANES), jnp.float32),
            jax.ShapeDtypeStruct((V, 8), jnp.float32),
        ],
    )


@functools.cache
def _make_sc_stage(B, L, V, D):
    info = plsc.get_sparse_core_info()
    NC, NS = info.num_cores, info.num_subcores
    NW = NC * NS
    NL = info.num_lanes
    assert B % NW == 0 and D == NL and L >= NL
    seq_per_w = B // NW
    koffs = [i * NL for i in range(L // NL)]
    if L % NL:
        koffs.append(L - NL)

    mesh = plsc.VectorSubcoreMesh(core_axis_name="c", subcore_axis_name="s")

    @functools.partial(
        pl.kernel,
        mesh=mesh,
        compiler_params=pltpu.CompilerParams(use_tc_tiling_on_sc=False,
                                             needs_layout_passes=False),
        out_type=jax.ShapeDtypeStruct((L, D, B), jnp.float32),
        scratch_types=[
            pltpu.VMEM((seq_per_w, LANES), jnp.int32),
            pltpu.VMEM((V, D), jnp.float32),
            pltpu.VMEM((seq_per_w * L, D), jnp.float32),
            pltpu.VMEM((L * D, seq_per_w), jnp.float32),
            pltpu.SemaphoreType.DMA,
        ],
    )
    def sc_kernel(ids_hbm, table_hbm, out_hbm, idx_v, tab_v, rows_v,
                  gvt_v, sem):
        wid = lax.axis_index("s") * NC + lax.axis_index("c")

        pltpu.sync_copy(ids_hbm.at[pl.ds(wid * seq_per_w, seq_per_w)], idx_v)
        pltpu.sync_copy(table_hbm.at[pl.ds(0, V), pl.ds(0, D)], tab_v)

        lane_i = jnp.arange(NL, dtype=jnp.int32)
        zeros_i = jnp.zeros((NL,), jnp.int32)
        dvecs = [jnp.full((NL,), d, jnp.int32) for d in range(D)]
        zero = jnp.zeros((D,), jnp.float32)
        PAIR = 2

        def seq_body(i, carry):
            seqs = [i * PAIR + j for j in range(PAIR)]
            svecs = [zeros_i + s for s in seqs]

            for s in seqs:
                for koff in koffs:
                    v = idx_v[s, pl.ds(koff, NL)]
                    tokvec = lane_i + (s * L + koff)
                    vals = [plsc.load_gather(tab_v, [v, dvecs[d]])
                            for d in range(D)]
                    for d in range(D):
                        plsc.store_scatter(rows_v, [tokvec, dvecs[d]],
                                           vals[d])

            es = [{} for _ in seqs]
            cums = [{0: zero} for _ in seqs]

            def emit(j, p):
                c = cums[j]
                if p < WINDOW:
                    g = c[p + WINDOW] - c[1]
                elif p + WINDOW > L:
                    g = c[L - 1] - c[p - WINDOW]
                else:
                    g = c[p + WINDOW] - c[p - WINDOW] - es[j][p]
                rowvec = lane_i + p * D
                plsc.store_scatter(gvt_v, [rowvec, svecs[j]], g)

            lag = WINDOW + 1
            for t in range(L):
                for j, s in enumerate(seqs):
                    e = rows_v[s * L + t, :]
                    es[j][t] = e
                    cums[j][t + 1] = cums[j][t] + e
                for j in range(PAIR):
                    p = t - lag
                    if p >= 0:
                        emit(j, p)
            for p in range(L - lag, L):
                for j in range(PAIR):
                    emit(j, p)
            return carry

        lax.fori_loop(0, seq_per_w // PAIR, seq_body, 0)

        copies = []
        for l in range(L):
            copies.append(
                pltpu.async_copy(
                    gvt_v.at[pl.ds(l * D, D)],
                    out_hbm.at[l, pl.ds(0, D),
                               pl.ds(wid * seq_per_w, seq_per_w)],
                    sem,
                )
            )
        for cp in copies:
            cp.wait()

    return sc_kernel


@functools.cache
def _make_tc_stage(B, L, V, D, LB=2):
    assert L % LB == 0

    def mm_body(x_ref, w_ref, b_ref, o_ref):
        w = w_ref[...]
        bias = b_ref[...][:, 0:1]
        for j in range(LB):
            o_ref[j] = (
                lax.dot_general(
                    w, x_ref[j],
                    (((0,), (0,)), ((), ())),
                    preferred_element_type=jnp.float32,
                )
                + bias
            )

    return pl.pallas_call(
        mm_body,
        grid=(L // LB,),
        in_specs=[
            pl.BlockSpec((LB, D, B), lambda i: (i, 0, 0)),
            pl.BlockSpec((D, V), lambda i: (0, 0)),
            pl.BlockSpec((V, 8), lambda i: (0, 0)),
        ],
        out_specs=pl.BlockSpec((LB, V, B), lambda i: (i, 0, 0)),
        out_shape=jax.ShapeDtypeStruct((L, V, B), jnp.float32),
    )


def kernel(ids, emb_table, W, b):
    B, L = ids.shape
    V, D = emb_table.shape
    ids = ids.astype(jnp.int32)
    ids_img, tab_pad, b_t = _make_prep_stage(B, L, V, D)(
        jnp.transpose(ids), jnp.transpose(emb_table), b)
    grouped_t = _make_sc_stage(B, L, V, D)(ids_img, tab_pad)
    out_t = _make_tc_stage(B, L, V, D)(grouped_t, jnp.transpose(W), b_t)
    return jnp.transpose(out_t, (2, 0, 1))


# speedup vs baseline: 1.0131x; 1.0033x over previous
import functools

import jax
import jax.numpy as jnp
from jax import lax
from jax.experimental import pallas as pl
from jax.experimental.pallas import tpu as pltpu
from jax.experimental.pallas import tpu_sc as plsc

WINDOW = 5
LANES = 128


@functools.cache
def _make_prep_stage(B, L, V, D):
    def prep_body(ids_ref, tab_ref, b_ref, idsimg_ref, tabpad_ref, bt_ref):
        idsimg_ref[...] = jnp.concatenate(
            [jnp.transpose(ids_ref[...]),
             jnp.zeros((B, LANES - L), jnp.int32)], axis=1)
        tabpad_ref[...] = jnp.concatenate(
            [jnp.transpose(tab_ref[...]),
             jnp.zeros((V, LANES - D), jnp.float32)], axis=1)
        bt_ref[...] = jnp.transpose(
            jnp.broadcast_to(b_ref[...].reshape(1, V), (8, V)))

    return pl.pallas_call(
        prep_body,
        out_shape=[
            jax.ShapeDtypeStruct((B, LANES), jnp.int32),
            jax.ShapeDtypeStruct((V, LANES), jnp.float32),
            jax.ShapeDtypeStruct((V, 8), jnp.float32),
        ],
    )


@functools.cache
def _make_sc_stage(B, L, V, D):
    info = plsc.get_sparse_core_info()
    NC, NS = info.num_cores, info.num_subcores
    NW = NC * NS
    NL = info.num_lanes
    assert B % NW == 0 and D == NL and L >= NL
    seq_per_w = B // NW
    koffs = [i * NL for i in range(L // NL)]
    if L % NL:
        koffs.append(L - NL)

    mesh = plsc.VectorSubcoreMesh(core_axis_name="c", subcore_axis_name="s")

    @functools.partial(
        pl.kernel,
        mesh=mesh,
        compiler_params=pltpu.CompilerParams(use_tc_tiling_on_sc=False,
                                             needs_layout_passes=False),
        out_type=jax.ShapeDtypeStruct((L, D, B), jnp.float32),
        scratch_types=[
            pltpu.VMEM((seq_per_w, LANES), jnp.int32),
            pltpu.VMEM((V, D), jnp.float32),
            pltpu.VMEM((seq_per_w * L, D), jnp.float32),
            pltpu.VMEM((L * D, seq_per_w), jnp.float32),
            pltpu.SemaphoreType.DMA,
        ],
    )
    def sc_kernel(ids_hbm, table_hbm, out_hbm, idx_v, tab_v, rows_v,
                  gvt_v, sem):
        wid = lax.axis_index("s") * NC + lax.axis_index("c")

        pltpu.sync_copy(ids_hbm.at[pl.ds(wid * seq_per_w, seq_per_w)], idx_v)
        pltpu.sync_copy(table_hbm.at[pl.ds(0, V), pl.ds(0, D)], tab_v)

        lane_i = jnp.arange(NL, dtype=jnp.int32)
        zeros_i = jnp.zeros((NL,), jnp.int32)
        dvecs = [jnp.full((NL,), d, jnp.int32) for d in range(D)]
        zero = jnp.zeros((D,), jnp.float32)
        PAIR = 2

        def seq_body(i, carry):
            seqs = [i * PAIR + j for j in range(PAIR)]
            svecs = [zeros_i + s for s in seqs]

            for s in seqs:
                for koff in koffs:
                    v = idx_v[s, pl.ds(koff, NL)]
                    tokvec = lane_i + (s * L + koff)
                    vals = [plsc.load_gather(tab_v, [v, dvecs[d]])
                            for d in range(D)]
                    for d in range(D):
                        plsc.store_scatter(rows_v, [tokvec, dvecs[d]],
                                           vals[d])

            es = [{} for _ in seqs]
            cums = [{0: zero} for _ in seqs]

            def emit(j, p):
                c = cums[j]
                if p < WINDOW:
                    g = c[p + WINDOW] - c[1]
                elif p + WINDOW > L:
                    g = c[L - 1] - c[p - WINDOW]
                else:
                    g = c[p + WINDOW] - c[p - WINDOW] - es[j][p]
                rowvec = lane_i + p * D
                plsc.store_scatter(gvt_v, [rowvec, svecs[j]], g)

            lag = WINDOW + 2
            for t in range(L):
                for j, s in enumerate(seqs):
                    e = rows_v[s * L + t, :]
                    es[j][t] = e
                    cums[j][t + 1] = cums[j][t] + e
                for j in range(PAIR):
                    p = t - lag
                    if p >= 0:
                        emit(j, p)
            for p in range(L - lag, L):
                for j in range(PAIR):
                    emit(j, p)
            return carry

        lax.fori_loop(0, seq_per_w // PAIR, seq_body, 0)

        copies = []
        for l in range(L):
            copies.append(
                pltpu.async_copy(
                    gvt_v.at[pl.ds(l * D, D)],
                    out_hbm.at[l, pl.ds(0, D),
                               pl.ds(wid * seq_per_w, seq_per_w)],
                    sem,
                )
            )
        for cp in copies:
            cp.wait()

    return sc_kernel


@functools.cache
def _make_tc_stage(B, L, V, D, LB=2):
    assert L % LB == 0

    def mm_body(x_ref, w_ref, b_ref, o_ref):
        w = w_ref[...]
        bias = b_ref[...][:, 0:1]
        for j in range(LB):
            o_ref[j] = (
                lax.dot_general(
                    w, x_ref[j],
                    (((0,), (0,)), ((), ())),
                    preferred_element_type=jnp.float32,
                )
                + bias
            )

    return pl.pallas_call(
        mm_body,
        grid=(L // LB,),
        in_specs=[
            pl.BlockSpec((LB, D, B), lambda i: (i, 0, 0)),
            pl.BlockSpec((D, V), lambda i: (0, 0)),
            pl.BlockSpec((V, 8), lambda i: (0, 0)),
        ],
        out_specs=pl.BlockSpec((LB, V, B), lambda i: (i, 0, 0)),
        out_shape=jax.ShapeDtypeStruct((L, V, B), jnp.float32),
    )


def kernel(ids, emb_table, W, b):
    B, L = ids.shape
    V, D = emb_table.shape
    ids = ids.astype(jnp.int32)
    ids_img, tab_pad, b_t = _make_prep_stage(B, L, V, D)(
        jnp.transpose(ids), jnp.transpose(emb_table), b)
    grouped_t = _make_sc_stage(B, L, V, D)(ids_img, tab_pad)
    out_t = _make_tc_stage(B, L, V, D)(grouped_t, jnp.transpose(W), b_t)
    return jnp.transpose(out_t, (2, 0, 1))
